# baseline (speedup 1.0000x reference)
import functools

import jax
import jax.numpy as jnp
from jax import lax
from jax.experimental import pallas as pl
from jax.experimental.pallas import tpu as pltpu
from jax.experimental.pallas import tpu_sc as plsc

N = 10000
E = 320000
NPAD = 10240
TOT = 2528
EC = TOT * 128
R = 640

NC0_L2, NC1_L2, SBUF_L2 = 80, 78, 80

_mesh = plsc.VectorSubcoreMesh(core_axis_name="c", subcore_axis_name="s")


HPAD = 16384


@functools.partial(
    pl.kernel,
    out_type=jax.ShapeDtypeStruct((32, HPAD), jnp.float32),
    mesh=_mesh,
    compiler_params=pltpu.CompilerParams(needs_layout_passes=False, use_tc_tiling_on_sc=False),
    scratch_types=[
        pltpu.VMEM((10000,), jnp.int32),
        pltpu.VMEM((HPAD,), jnp.float32),
    ],
)
def _deg_kernel(dst_hbm, out, dst_v, hist_v):
    c = lax.axis_index("c")
    s = lax.axis_index("s")
    wid = c * 16 + s

    def zrow(i, carry):
        hist_v[pl.ds(i * 16, 16)] = jnp.zeros((16,), jnp.float32)
        return carry

    lax.fori_loop(0, HPAD // 16, zrow, 0)

    pltpu.sync_copy(dst_hbm.at[wid], dst_v)

    ones = jnp.ones((16,), jnp.float32)

    def accum(i, carry):
        idx = dst_v[pl.ds(i * 16, 16)]
        plsc.addupdate_scatter(hist_v, [idx], ones)
        return carry

    lax.fori_loop(0, 625, accum, 0)
    pltpu.sync_copy(hist_v, out.at[wid])


def _dual_feature_agg(D, sbuf):
    nchunk = TOT // 16

    def _phases(nc):
        full, rem = divmod(nc, sbuf)
        return [sbuf] * full + ([rem] if rem else [])

    @functools.partial(
        pl.kernel,
        out_type=jax.ShapeDtypeStruct((2, NPAD, D), jnp.float32),
        mesh=_mesh,
        compiler_params=pltpu.CompilerParams(needs_layout_passes=False, use_tc_tiling_on_sc=False),
        scratch_types=[
            pltpu.VMEM((sbuf, 128), jnp.int32),
            pltpu.VMEM((sbuf, 128), jnp.int32),
            pltpu.VMEM((128, D), jnp.float32),
            pltpu.VMEM((128, D), jnp.float32),
            pltpu.VMEM_SHARED((NPAD, D), jnp.float32),
            pltpu.VMEM_SHARED((NPAD, D), jnp.float32),
            pltpu.SemaphoreType.DMA,
            pltpu.SemaphoreType.DMA,
            pltpu.SemaphoreType.DMA,
            pltpu.SemaphoreType.DMA,
        ],
    )
    def agg(table_hbm, src_hbm, dst_hbm, out,
            src_v, dst_v, rb0, rb1, tab_sh, acc_sh, sem0, sem1, sem2, sem3):
        c = lax.axis_index("c")
        s = lax.axis_index("s")
        table = table_hbm.at[c]

        pltpu.sync_copy(table.at[pl.ds(s * 640, 640)],
                        tab_sh.at[pl.ds(s * 640, 640)])

        def zrow(i, carry):
            def zcol(k, inner):
                rb0[i, pl.ds(k * 16, 16)] = jnp.zeros((16,), jnp.float32)
                return inner
            return lax.fori_loop(0, D // 16, zcol, carry)

        lax.fori_loop(0, 128, zrow, 0)
        for j in range(5):
            pltpu.sync_copy(rb0, acc_sh.at[pl.ds(s * 640 + j * 128, 128)])
        plsc.subcore_barrier()

        def start_sp(j):
            pltpu.make_async_copy(tab_sh.at[src_v.at[j]], rb0, sem0).start()

        def start_hbm(j):
            pltpu.make_async_copy(table.at[src_v.at[j]], rb1, sem1).start()

        def wait(rb, sem):
            pltpu.make_async_copy(table.at[src_v.at[0]], rb, sem).wait()

        off = 0
        for pc in _phases(nchunk):
            pltpu.sync_copy(src_hbm.at[pl.ds(s * nchunk + off, pc)],
                            src_v.at[pl.ds(0, pc)])
            pltpu.sync_copy(dst_hbm.at[pl.ds(s * nchunk + off, pc)],
                            dst_v.at[pl.ds(0, pc)])
            off += pc

            start_sp(0)
            start_hbm(1)

            def body(j2, carry):
                j = j2 * 2
                wait(rb0, sem0)
                pltpu.async_copy(rb0, acc_sh.at[dst_v.at[j]], sem2, add=True)
                wait(rb1, sem1)
                pltpu.async_copy(rb1, acc_sh.at[dst_v.at[j + 1]], sem3, add=True)

                pltpu.make_async_copy(rb0, acc_sh.at[dst_v.at[j]], sem2).wait()

                @pl.when(j + 2 < pc)
                def _():
                    start_sp(j + 2)

                pltpu.make_async_copy(rb1, acc_sh.at[dst_v.at[j + 1]], sem3).wait()

                @pl.when(j + 3 < pc)
                def _():
                    start_hbm(j + 3)

                return carry

            lax.fori_loop(0, pc // 2, body, 0)

        plsc.subcore_barrier()
        pltpu.sync_copy(acc_sh.at[pl.ds(s * 640, 640)],
                        out.at[c].at[pl.ds(s * 640, 640)])

    return agg


def _dual_edge_agg(D, nc0, nc1, sbuf):
    assert 16 * (nc0 + nc1) == TOT and nc0 % 2 == 0 and nc1 % 2 == 0

    def _phases(nc):
        full, rem = divmod(nc, sbuf)
        return [sbuf] * full + ([rem] if rem else [])

    @functools.partial(
        pl.kernel,
        out_type=jax.ShapeDtypeStruct((2, NPAD, D), jnp.float32),
        mesh=_mesh,
        compiler_params=pltpu.CompilerParams(needs_layout_passes=False, use_tc_tiling_on_sc=False),
        scratch_types=[
            pltpu.VMEM((sbuf, 128), jnp.int32),
            pltpu.VMEM((sbuf, 128), jnp.int32),
            pltpu.VMEM((128, D), jnp.float32),
            pltpu.VMEM((128, D), jnp.float32),
            pltpu.VMEM_SHARED((NPAD, D), jnp.float32),
            pltpu.VMEM_SHARED((NPAD, D), jnp.float32),
            pltpu.SemaphoreType.DMA,
            pltpu.SemaphoreType.DMA,
            pltpu.SemaphoreType.DMA,
            pltpu.SemaphoreType.DMA,
        ],
    )
    def agg(table_hbm, src_hbm, dst_hbm, out,
            src_v, dst_v, rb0, rb1, tab_sh, acc_sh, sem0, sem1, sem2, sem3):
        c = lax.axis_index("c")
        s = lax.axis_index("s")

        pltpu.sync_copy(table_hbm.at[pl.ds(s * 640, 640)],
                        tab_sh.at[pl.ds(s * 640, 640)])

        def zrow(i, carry):
            def zcol(k, inner):
                rb0[i, pl.ds(k * 16, 16)] = jnp.zeros((16,), jnp.float32)
                return inner
            return lax.fori_loop(0, D // 16, zcol, carry)

        lax.fori_loop(0, 128, zrow, 0)
        for j in range(5):
            pltpu.sync_copy(rb0, acc_sh.at[pl.ds(s * 640 + j * 128, 128)])
        plsc.subcore_barrier()

        def start_sp(j):
            pltpu.make_async_copy(tab_sh.at[src_v.at[j]], rb0, sem0).start()

        def start_hbm(j):
            pltpu.make_async_copy(table_hbm.at[src_v.at[j]], rb1, sem1).start()

        def wait(rb, sem):
            pltpu.make_async_copy(table_hbm.at[src_v.at[0]], rb, sem).wait()

        def run(base, nc):
            off = 0
            for pc in _phases(nc):
                pltpu.sync_copy(src_hbm.at[pl.ds(base + off, pc)],
                                src_v.at[pl.ds(0, pc)])
                pltpu.sync_copy(dst_hbm.at[pl.ds(base + off, pc)],
                                dst_v.at[pl.ds(0, pc)])
                off += pc

                start_sp(0)
                start_hbm(1)

                def body(j2, carry):
                    j = j2 * 2
                    wait(rb0, sem0)
                    pltpu.async_copy(rb0, acc_sh.at[dst_v.at[j]], sem2, add=True)
                    wait(rb1, sem1)
                    pltpu.async_copy(rb1, acc_sh.at[dst_v.at[j + 1]], sem3, add=True)

                    pltpu.make_async_copy(rb0, acc_sh.at[dst_v.at[j]], sem2).wait()

                    @pl.when(j + 2 < pc)
                    def _():
                        start_sp(j + 2)

                    pltpu.make_async_copy(rb1, acc_sh.at[dst_v.at[j + 1]], sem3).wait()

                    @pl.when(j + 3 < pc)
                    def _():
                        start_hbm(j + 3)

                    return carry

                lax.fori_loop(0, pc // 2, body, 0)

        @pl.when(c == 1)
        def _():
            run(s * nc1, nc1)

        @pl.when(c == 0)
        def _():
            run(16 * nc1 + s * nc0, nc0)

        plsc.subcore_barrier()
        pltpu.sync_copy(acc_sh.at[pl.ds(s * 640, 640)],
                        out.at[c].at[pl.ds(s * 640, 640)])

    return agg


_agg64 = _dual_feature_agg(64, sbuf=80)
_agg32 = _dual_edge_agg(32, NC0_L2, NC1_L2, SBUF_L2)


def _tc_b_body(x_ref, w_ref, cnt_ref, hs_ref, disp_ref):
    deg = jnp.sum(cnt_ref[...], axis=0)[:, None] + 1.0
    dis = lax.rsqrt(deg)
    h = jnp.dot(x_ref[...] * dis, w_ref[...], preferred_element_type=jnp.float32)
    hs_ref[0] = h[:, :64]
    hs_ref[1] = h[:, 64:]
    disp_ref[...] = jnp.broadcast_to(dis, (R, 2))


_tc_b = pl.pallas_call(
    _tc_b_body,
    grid=(NPAD // R,),
    in_specs=[
        pl.BlockSpec((R, 128), lambda i: (i, 0)),
        pl.BlockSpec((128, 128), lambda i: (0, 0)),
        pl.BlockSpec((32, R), lambda i: (0, i)),
    ],
    out_specs=[
        pl.BlockSpec((2, R, 64), lambda i: (0, i, 0)),
        pl.BlockSpec((R, 2), lambda i: (i, 0)),
    ],
    out_shape=[
        jax.ShapeDtypeStruct((2, NPAD, 64), jnp.float32),
        jax.ShapeDtypeStruct((NPAD, 2), jnp.float32),
    ],
)


def _leaky(a):
    return jnp.where(a >= 0, a, 0.01 * a)


def _tc_d_body(seg_ref, hs_ref, disp_ref, b1_ref, w2_ref, out_ref):
    dis = disp_ref[...][:, :1]
    a = jnp.concatenate([seg_ref[0] + hs_ref[0], seg_ref[1] + hs_ref[1]],
                        axis=1) * dis + b1_ref[...]
    out_ref[...] = jnp.dot(_leaky(a), w2_ref[...],
                           preferred_element_type=jnp.float32) * dis


_tc_d = pl.pallas_call(
    _tc_d_body,
    grid=(NPAD // R,),
    in_specs=[
        pl.BlockSpec((2, R, 64), lambda i: (0, i, 0)),
        pl.BlockSpec((2, R, 64), lambda i: (0, i, 0)),
        pl.BlockSpec((R, 2), lambda i: (i, 0)),
        pl.BlockSpec((1, 128), lambda i: (0, 0)),
        pl.BlockSpec((128, 32), lambda i: (0, 0)),
    ],
    out_specs=pl.BlockSpec((R, 32), lambda i: (i, 0)),
    out_shape=jax.ShapeDtypeStruct((NPAD, 32), jnp.float32),
)


def _tc_f_body(q_ref, hs_ref, disp_ref, b2_ref, wl_ref, bl_ref, out_ref):
    dis = disp_ref[...][:, :1]
    a = dis * (q_ref[0] + q_ref[1] + hs_ref[...]) + b2_ref[...]
    logits = jnp.dot(_leaky(a), wl_ref[...],
                     preferred_element_type=jnp.float32) + bl_ref[...]
    m = jnp.max(logits, axis=-1, keepdims=True)
    e = jnp.exp(logits - m)
    out_ref[...] = e / jnp.sum(e, axis=-1, keepdims=True)


_tc_f = pl.pallas_call(
    _tc_f_body,
    grid=(NPAD // R,),
    in_specs=[
        pl.BlockSpec((2, R, 32), lambda i: (0, i, 0)),
        pl.BlockSpec((R, 32), lambda i: (i, 0)),
        pl.BlockSpec((R, 2), lambda i: (i, 0)),
        pl.BlockSpec((1, 32), lambda i: (0, 0)),
        pl.BlockSpec((32, 2), lambda i: (0, 0)),
        pl.BlockSpec((1, 2), lambda i: (0, 0)),
    ],
    out_specs=pl.BlockSpec((R, 2), lambda i: (i, 0)),
    out_shape=jax.ShapeDtypeStruct((NPAD, 2), jnp.float32),
)


def kernel(x, edge_index, W1, b1, W2, b2, Wl, bl):
    src = edge_index[0].astype(jnp.int32)
    dst = edge_index[1].astype(jnp.int32)

    dst32 = dst.reshape(32, 10000)
    pad = jnp.full((EC - E,), N, jnp.int32)
    srcT = jnp.concatenate([src, pad]).reshape(TOT, 128)
    dstT = jnp.concatenate([dst, pad]).reshape(TOT, 128)
    x_pad = jnp.zeros((NPAD, 128), jnp.float32).at[:N].set(x)

    cnt32 = _deg_kernel(dst32)

    hs1, disp = _tc_b(x_pad, W1, cnt32)
    seg1 = _agg64(hs1, srcT, dstT)
    hs2 = _tc_d(seg1, hs1, disp, b1.reshape(1, 128), W2)
    q = _agg32(hs2, srcT, dstT)
    out = _tc_f(q, hs2, disp, b2.reshape(1, 32), Wl, bl.reshape(1, 2))
    return out[:N]


# speedup vs baseline: 1.0143x; 1.0143x over previous
import functools

import jax
import jax.numpy as jnp
from jax import lax
from jax.experimental import pallas as pl
from jax.experimental.pallas import tpu as pltpu
from jax.experimental.pallas import tpu_sc as plsc

N = 10000
E = 320000
NPAD = 10240
TOT = 2528
EC = TOT * 128
R = 640

NC0_L2, NC1_L2, SBUF_L2 = 80, 78, 80

_mesh = plsc.VectorSubcoreMesh(core_axis_name="c", subcore_axis_name="s")


HPAD = 16384


@functools.partial(
    pl.kernel,
    out_type=jax.ShapeDtypeStruct((32, HPAD), jnp.float32),
    mesh=_mesh,
    compiler_params=pltpu.CompilerParams(needs_layout_passes=False, use_tc_tiling_on_sc=False),
    scratch_types=[
        pltpu.VMEM((10000,), jnp.int32),
        pltpu.VMEM((HPAD,), jnp.float32),
    ],
)
def _deg_kernel(dst_hbm, out, dst_v, hist_v):
    c = lax.axis_index("c")
    s = lax.axis_index("s")
    wid = c * 16 + s

    def zrow(i, carry):
        hist_v[pl.ds(i * 16, 16)] = jnp.zeros((16,), jnp.float32)
        return carry

    lax.fori_loop(0, HPAD // 16, zrow, 0)

    pltpu.sync_copy(dst_hbm.at[wid], dst_v)

    ones = jnp.ones((16,), jnp.float32)

    def accum(i, carry):
        idx = dst_v[pl.ds(i * 16, 16)]
        plsc.addupdate_scatter(hist_v, [idx], ones)
        return carry

    lax.fori_loop(0, 625, accum, 0)
    pltpu.sync_copy(hist_v, out.at[wid])


def _dual_feature_agg(D, sbuf):
    nchunk = TOT // 16

    def _phases(nc):
        full, rem = divmod(nc, sbuf)
        return [sbuf] * full + ([rem] if rem else [])

    @functools.partial(
        pl.kernel,
        out_type=jax.ShapeDtypeStruct((2, NPAD, D), jnp.float32),
        mesh=_mesh,
        compiler_params=pltpu.CompilerParams(needs_layout_passes=False, use_tc_tiling_on_sc=False),
        scratch_types=[
            pltpu.VMEM((sbuf, 128), jnp.int32),
            pltpu.VMEM((sbuf, 128), jnp.int32),
            pltpu.VMEM((128, D), jnp.float32),
            pltpu.VMEM((128, D), jnp.float32),
            pltpu.VMEM_SHARED((NPAD, D), jnp.float32),
            pltpu.VMEM_SHARED((NPAD, D), jnp.float32),
            pltpu.SemaphoreType.DMA,
            pltpu.SemaphoreType.DMA,
        ],
    )
    def agg(table_hbm, src_hbm, dst_hbm, out,
            src_v, dst_v, rb0, rb1, tab_sh, acc_sh, sem0, sem1):
        c = lax.axis_index("c")
        s = lax.axis_index("s")
        table = table_hbm.at[c]

        pltpu.sync_copy(table.at[pl.ds(s * 640, 640)],
                        tab_sh.at[pl.ds(s * 640, 640)])

        def zrow(i, carry):
            def zcol(k, inner):
                rb0[i, pl.ds(k * 16, 16)] = jnp.zeros((16,), jnp.float32)
                return inner
            return lax.fori_loop(0, D // 16, zcol, carry)

        lax.fori_loop(0, 128, zrow, 0)
        for j in range(5):
            pltpu.sync_copy(rb0, acc_sh.at[pl.ds(s * 640 + j * 128, 128)])
        plsc.subcore_barrier()

        def start_sp(j):
            pltpu.make_async_copy(tab_sh.at[src_v.at[j]], rb0, sem0).start()

        def start_hbm(j):
            pltpu.make_async_copy(table.at[src_v.at[j]], rb1, sem1).start()

        def wait(rb, sem):
            pltpu.make_async_copy(table.at[src_v.at[0]], rb, sem).wait()

        off = 0
        for pc in _phases(nchunk):
            pltpu.sync_copy(src_hbm.at[pl.ds(s * nchunk + off, pc)],
                            src_v.at[pl.ds(0, pc)])
            pltpu.sync_copy(dst_hbm.at[pl.ds(s * nchunk + off, pc)],
                            dst_v.at[pl.ds(0, pc)])
            off += pc

            start_sp(0)
            start_hbm(1)

            def body(j2, carry):
                j = j2 * 2
                wait(rb0, sem0)
                pltpu.sync_copy(rb0, acc_sh.at[dst_v.at[j]], add=True)

                @pl.when(j + 2 < pc)
                def _():
                    start_sp(j + 2)

                wait(rb1, sem1)
                pltpu.sync_copy(rb1, acc_sh.at[dst_v.at[j + 1]], add=True)

                @pl.when(j + 3 < pc)
                def _():
                    start_hbm(j + 3)

                return carry

            lax.fori_loop(0, pc // 2, body, 0)

        plsc.subcore_barrier()
        pltpu.sync_copy(acc_sh.at[pl.ds(s * 640, 640)],
                        out.at[c].at[pl.ds(s * 640, 640)])

    return agg


def _dual_edge_agg(D, nc0, nc1, sbuf):
    assert 16 * (nc0 + nc1) == TOT and nc0 % 2 == 0 and nc1 % 2 == 0

    def _phases(nc):
        full, rem = divmod(nc, sbuf)
        return [sbuf] * full + ([rem] if rem else [])

    @functools.partial(
        pl.kernel,
        out_type=jax.ShapeDtypeStruct((2, NPAD, D), jnp.float32),
        mesh=_mesh,
        compiler_params=pltpu.CompilerParams(needs_layout_passes=False, use_tc_tiling_on_sc=False),
        scratch_types=[
            pltpu.VMEM((sbuf, 128), jnp.int32),
            pltpu.VMEM((sbuf, 128), jnp.int32),
            pltpu.VMEM((128, D), jnp.float32),
            pltpu.VMEM((128, D), jnp.float32),
            pltpu.VMEM_SHARED((NPAD, D), jnp.float32),
            pltpu.VMEM_SHARED((NPAD, D), jnp.float32),
            pltpu.SemaphoreType.DMA,
            pltpu.SemaphoreType.DMA,
        ],
    )
    def agg(table_hbm, src_hbm, dst_hbm, out,
            src_v, dst_v, rb0, rb1, tab_sh, acc_sh, sem0, sem1):
        c = lax.axis_index("c")
        s = lax.axis_index("s")

        pltpu.sync_copy(table_hbm.at[pl.ds(s * 640, 640)],
                        tab_sh.at[pl.ds(s * 640, 640)])

        def zrow(i, carry):
            def zcol(k, inner):
                rb0[i, pl.ds(k * 16, 16)] = jnp.zeros((16,), jnp.float32)
                return inner
            return lax.fori_loop(0, D // 16, zcol, carry)

        lax.fori_loop(0, 128, zrow, 0)
        for j in range(5):
            pltpu.sync_copy(rb0, acc_sh.at[pl.ds(s * 640 + j * 128, 128)])
        plsc.subcore_barrier()

        def start_sp(j):
            pltpu.make_async_copy(tab_sh.at[src_v.at[j]], rb0, sem0).start()

        def start_hbm(j):
            pltpu.make_async_copy(table_hbm.at[src_v.at[j]], rb1, sem1).start()

        def wait(rb, sem):
            pltpu.make_async_copy(table_hbm.at[src_v.at[0]], rb, sem).wait()

        def run(base, nc):
            off = 0
            for pc in _phases(nc):
                pltpu.sync_copy(src_hbm.at[pl.ds(base + off, pc)],
                                src_v.at[pl.ds(0, pc)])
                pltpu.sync_copy(dst_hbm.at[pl.ds(base + off, pc)],
                                dst_v.at[pl.ds(0, pc)])
                off += pc

                start_sp(0)
                start_hbm(1)

                def body(j2, carry):
                    j = j2 * 2
                    wait(rb0, sem0)
                    pltpu.sync_copy(rb0, acc_sh.at[dst_v.at[j]], add=True)

                    @pl.when(j + 2 < pc)
                    def _():
                        start_sp(j + 2)

                    wait(rb1, sem1)
                    pltpu.sync_copy(rb1, acc_sh.at[dst_v.at[j + 1]], add=True)

                    @pl.when(j + 3 < pc)
                    def _():
                        start_hbm(j + 3)

                    return carry

                lax.fori_loop(0, pc // 2, body, 0)

        @pl.when(c == 1)
        def _():
            run(s * nc1, nc1)

        @pl.when(c == 0)
        def _():
            run(16 * nc1 + s * nc0, nc0)

        plsc.subcore_barrier()
        pltpu.sync_copy(acc_sh.at[pl.ds(s * 640, 640)],
                        out.at[c].at[pl.ds(s * 640, 640)])

    return agg


_agg64 = _dual_feature_agg(64, sbuf=80)
_agg32 = _dual_edge_agg(32, NC0_L2, NC1_L2, SBUF_L2)


def _tc_b_body(x_ref, w_ref, cnt_ref, hs_ref, disp_ref):
    deg = jnp.sum(cnt_ref[...], axis=0)[:, None] + 1.0
    dis = lax.rsqrt(deg)
    h = jnp.dot(x_ref[...] * dis, w_ref[...], preferred_element_type=jnp.float32)
    hs_ref[0] = h[:, :64]
    hs_ref[1] = h[:, 64:]
    disp_ref[...] = jnp.broadcast_to(dis, (R, 2))


_tc_b = pl.pallas_call(
    _tc_b_body,
    grid=(NPAD // R,),
    in_specs=[
        pl.BlockSpec((R, 128), lambda i: (i, 0)),
        pl.BlockSpec((128, 128), lambda i: (0, 0)),
        pl.BlockSpec((32, R), lambda i: (0, i)),
    ],
    out_specs=[
        pl.BlockSpec((2, R, 64), lambda i: (0, i, 0)),
        pl.BlockSpec((R, 2), lambda i: (i, 0)),
    ],
    out_shape=[
        jax.ShapeDtypeStruct((2, NPAD, 64), jnp.float32),
        jax.ShapeDtypeStruct((NPAD, 2), jnp.float32),
    ],
)


def _leaky(a):
    return jnp.where(a >= 0, a, 0.01 * a)


def _tc_d_body(seg_ref, hs_ref, disp_ref, b1_ref, w2_ref, out_ref):
    dis = disp_ref[...][:, :1]
    a = jnp.concatenate([seg_ref[0] + hs_ref[0], seg_ref[1] + hs_ref[1]],
                        axis=1) * dis + b1_ref[...]
    out_ref[...] = jnp.dot(_leaky(a), w2_ref[...],
                           preferred_element_type=jnp.float32) * dis


_tc_d = pl.pallas_call(
    _tc_d_body,
    grid=(NPAD // R,),
    in_specs=[
        pl.BlockSpec((2, R, 64), lambda i: (0, i, 0)),
        pl.BlockSpec((2, R, 64), lambda i: (0, i, 0)),
        pl.BlockSpec((R, 2), lambda i: (i, 0)),
        pl.BlockSpec((1, 128), lambda i: (0, 0)),
        pl.BlockSpec((128, 32), lambda i: (0, 0)),
    ],
    out_specs=pl.BlockSpec((R, 32), lambda i: (i, 0)),
    out_shape=jax.ShapeDtypeStruct((NPAD, 32), jnp.float32),
)


def _tc_f_body(q_ref, hs_ref, disp_ref, b2_ref, wl_ref, bl_ref, out_ref):
    dis = disp_ref[...][:, :1]
    a = dis * (q_ref[0] + q_ref[1] + hs_ref[...]) + b2_ref[...]
    logits = jnp.dot(_leaky(a), wl_ref[...],
                     preferred_element_type=jnp.float32) + bl_ref[...]
    m = jnp.max(logits, axis=-1, keepdims=True)
    e = jnp.exp(logits - m)
    out_ref[...] = e / jnp.sum(e, axis=-1, keepdims=True)


_tc_f = pl.pallas_call(
    _tc_f_body,
    grid=(NPAD // R,),
    in_specs=[
        pl.BlockSpec((2, R, 32), lambda i: (0, i, 0)),
        pl.BlockSpec((R, 32), lambda i: (i, 0)),
        pl.BlockSpec((R, 2), lambda i: (i, 0)),
        pl.BlockSpec((1, 32), lambda i: (0, 0)),
        pl.BlockSpec((32, 2), lambda i: (0, 0)),
        pl.BlockSpec((1, 2), lambda i: (0, 0)),
    ],
    out_specs=pl.BlockSpec((R, 2), lambda i: (i, 0)),
    out_shape=jax.ShapeDtypeStruct((NPAD, 2), jnp.float32),
)


def kernel(x, edge_index, W1, b1, W2, b2, Wl, bl):
    src = edge_index[0].astype(jnp.int32)
    dst = edge_index[1].astype(jnp.int32)

    dst32 = dst.reshape(32, 10000)
    pad = jnp.full((EC - E,), N, jnp.int32)
    srcT = jnp.concatenate([src, pad]).reshape(TOT, 128)
    dstT = jnp.concatenate([dst, pad]).reshape(TOT, 128)
    x_pad = jnp.zeros((NPAD, 128), jnp.float32).at[:N].set(x)

    cnt32 = _deg_kernel(dst32)

    hs1, disp = _tc_b(x_pad, W1, cnt32)
    seg1 = _agg64(hs1, srcT, dstT)
    hs2 = _tc_d(seg1, hs1, disp, b1.reshape(1, 128), W2)
    q = _agg32(hs2, srcT, dstT)
    out = _tc_f(q, hs2, disp, b2.reshape(1, 32), Wl, bl.reshape(1, 2))
    return out[:N]
